# baseline (speedup 1.0000x reference)
import functools

import jax
import jax.numpy as jnp
from jax.experimental import pallas as pl
from jax.experimental.pallas import tpu as pltpu

B, T, V, D = 16, 50, 64, 768
M = 64
POSF = 7
MAX_STEPS = 100
EPS = 1e-12
TV = T * V
C = 5
TK = T // C
RK = TK * V


def _encoder_kernel(x_ref, lens_ref, vpids_ref, mask_ref, cand_ref, sid_ref,
                    pos_ref, wpos_ref, bpos_ref, gam_ref, bet_ref, table_ref,
                    out_ref, cand_acc, vis_acc, cnt_acc):
    f32 = jnp.float32
    c = pl.program_id(1)

    x = x_ref[0, 0]
    mask_col = mask_ref[0, 0]
    cand = cand_ref[0, 0]
    m_iota = jax.lax.broadcasted_iota(jnp.int32, (RK, M), 1)
    onehot = jnp.where(cand == m_iota, mask_col, 0.0)
    part = jax.lax.dot_general(
        onehot, x, (((0,), (0,)), ((), ())),
        preferred_element_type=f32)
    part_cnt = jnp.sum(onehot, axis=0, keepdims=True)

    masked = x * mask_col
    step_part = jnp.sum(masked.reshape(TK, V, D), axis=1)

    vp = vpids_ref[0]
    m_iota_t = jax.lax.broadcasted_iota(jnp.int32, (T, M), 1)
    t_iota = jax.lax.broadcasted_iota(jnp.int32, (T, M), 0)
    hit = (vp + 1) == m_iota_t
    tstar = jnp.max(jnp.where(hit, t_iota + 1, 0), axis=0, keepdims=True)
    tstar_col = tstar.reshape(M, 1)
    tloc = tstar_col - 1 - c * TK
    j_iota = jax.lax.broadcasted_iota(jnp.int32, (M, TK), 1)
    vis_c = jnp.where((tloc == j_iota) & (tstar_col > 0), 1.0, 0.0)
    vis_part = jnp.dot(vis_c, step_part, preferred_element_type=f32)

    @pl.when(c == 0)
    def _():
        cand_acc[...] = part
        vis_acc[...] = vis_part
        cnt_acc[...] = part_cnt

    @pl.when(c > 0)
    def _():
        cand_acc[...] += part
        vis_acc[...] += vis_part
        cnt_acc[...] += part_cnt

    @pl.when(c == C - 1)
    def _():
        lensf = jnp.maximum(lens_ref[0], 1).astype(f32)
        unvisited = cand_acc[...] / jnp.maximum(
            cnt_acc[...].reshape(M, 1), 1.0)
        t_iota_m = jax.lax.broadcasted_iota(jnp.int32, (M, T), 1)
        onehot_vis = ((tstar_col - 1) == t_iota_m).astype(f32)
        len_sel = jnp.dot(onehot_vis, lensf, preferred_element_type=f32)
        visited_fts = vis_acc[...] / jnp.maximum(len_sel, 1.0)
        vis_mask = tstar_col > 0
        img = jnp.where(vis_mask, visited_fts, unvisited)
        node_iota = jax.lax.broadcasted_iota(jnp.int32, (M, 1), 0)
        img = jnp.where(node_iota == 0, 0.0, img)

        sid = sid_ref[0]
        s_iota = jax.lax.broadcasted_iota(jnp.int32, (M, MAX_STEPS), 1)
        onehot_step = (sid == s_iota).astype(f32)
        step_emb = jnp.dot(onehot_step, table_ref[...],
                           preferred_element_type=f32)

        h = jnp.dot(pos_ref[0], wpos_ref[...],
                    preferred_element_type=f32) + bpos_ref[...]
        mu = jnp.mean(h, axis=1, keepdims=True)
        var = jnp.mean((h - mu) ** 2, axis=1, keepdims=True)
        ln = (h - mu) / jnp.sqrt(var + EPS) * gam_ref[...] + bet_ref[...]

        out_ref[0] = img + step_emb + ln


@jax.jit
def _encode(split_traj_embeds, split_traj_vp_lens, traj_vpids, traj_cand_vpids,
            gmap_step_ids, gmap_pos_fts, W_pos, b_pos, ln_gamma, ln_beta,
            step_table):
    x = split_traj_embeds.reshape(B, C, RK, D)
    lens = split_traj_vp_lens.reshape(B, T, 1)
    vpids = traj_vpids.reshape(B, T, 1)
    lens_c = jnp.maximum(split_traj_vp_lens, 1)
    mask_flat = (jnp.arange(V)[None, None, :] < lens_c[:, :, None]).astype(
        jnp.float32).reshape(B, C, RK, 1)
    cand_flat = traj_cand_vpids.reshape(B, C, RK, 1)
    sid = gmap_step_ids.reshape(B, M, 1)
    pos = jnp.pad(gmap_pos_fts, ((0, 0), (0, 0), (0, 8 - POSF)))
    wpos = jnp.pad(W_pos, ((0, 8 - POSF), (0, 0)))
    bpos = b_pos.reshape(1, D)
    gam = ln_gamma.reshape(1, D)
    bet = ln_beta.reshape(1, D)

    out = pl.pallas_call(
        _encoder_kernel,
        grid=(B, C),
        in_specs=[
            pl.BlockSpec((1, 1, RK, D), lambda b, c: (b, c, 0, 0)),
            pl.BlockSpec((1, T, 1), lambda b, c: (b, 0, 0)),
            pl.BlockSpec((1, T, 1), lambda b, c: (b, 0, 0)),
            pl.BlockSpec((1, 1, RK, 1), lambda b, c: (b, c, 0, 0)),
            pl.BlockSpec((1, 1, RK, 1), lambda b, c: (b, c, 0, 0)),
            pl.BlockSpec((1, M, 1), lambda b, c: (b, 0, 0)),
            pl.BlockSpec((1, M, 8), lambda b, c: (b, 0, 0)),
            pl.BlockSpec((8, D), lambda b, c: (0, 0)),
            pl.BlockSpec((1, D), lambda b, c: (0, 0)),
            pl.BlockSpec((1, D), lambda b, c: (0, 0)),
            pl.BlockSpec((1, D), lambda b, c: (0, 0)),
            pl.BlockSpec((MAX_STEPS, D), lambda b, c: (0, 0)),
        ],
        out_specs=pl.BlockSpec((1, M, D), lambda b, c: (b, 0, 0)),
        out_shape=jax.ShapeDtypeStruct((B, M, D), jnp.float32),
        scratch_shapes=[
            pltpu.VMEM((M, D), jnp.float32),
            pltpu.VMEM((M, D), jnp.float32),
            pltpu.VMEM((1, M), jnp.float32),
        ],
    )(x, lens, vpids, mask_flat, cand_flat, sid, pos, wpos, bpos, gam, bet,
      step_table)
    return out


def kernel(txt_embeds, txt_masks, split_traj_embeds, split_traj_vp_lens,
           traj_vpids, traj_cand_vpids, gmap_vpids, gmap_step_ids,
           gmap_pos_fts, gmap_lens, W_pos, b_pos, ln_gamma, ln_beta,
           step_table):
    return _encode(split_traj_embeds, split_traj_vp_lens, traj_vpids,
                   traj_cand_vpids, gmap_step_ids, gmap_pos_fts, W_pos, b_pos,
                   ln_gamma, ln_beta, step_table)


# speedup vs baseline: 1.2664x; 1.2664x over previous
import functools

import jax
import jax.numpy as jnp
from jax.experimental import pallas as pl
from jax.experimental.pallas import tpu as pltpu

B, T, V, D = 16, 50, 64, 768
M = 64
POSF = 7
MAX_STEPS = 100
EPS = 1e-12
TV = T * V


def _encoder_kernel(x_ref, lens_ref, vpids_ref, mask_ref, cand_ref, sid_ref,
                    pos_ref, wpos_ref, bpos_ref, gam_ref, bet_ref, table_ref,
                    out_ref):
    f32 = jnp.float32
    x = x_ref[0]
    lens = jnp.maximum(lens_ref[0], 1)
    lensf = lens.astype(f32)
    mask_col = mask_ref[0]

    cand = cand_ref[0]
    m_iota = jax.lax.broadcasted_iota(jnp.int32, (TV, M), 1)
    onehot = jnp.where(cand == m_iota, mask_col, 0.0)
    cand_sum = jax.lax.dot_general(
        onehot, x, (((0,), (0,)), ((), ())),
        preferred_element_type=f32)
    cnt = jnp.sum(onehot, axis=0, keepdims=True)
    unvisited = cand_sum / jnp.maximum(cnt.reshape(M, 1), 1.0)

    masked = x * mask_col
    step_sum = jnp.sum(masked.reshape(T, V, D), axis=1)
    step_fts = step_sum / lensf

    vp = vpids_ref[0]
    m_iota_t = jax.lax.broadcasted_iota(jnp.int32, (T, M), 1)
    t_iota = jax.lax.broadcasted_iota(jnp.int32, (T, M), 0)
    hit = (vp + 1) == m_iota_t
    tstar = jnp.max(jnp.where(hit, t_iota + 1, 0), axis=0, keepdims=True)
    tstar_col = tstar.reshape(M, 1)
    vis_mask = tstar_col > 0
    t_iota_m = jax.lax.broadcasted_iota(jnp.int32, (M, T), 1)
    onehot_vis = ((tstar_col - 1) == t_iota_m).astype(f32)
    visited_fts = jnp.dot(onehot_vis, step_fts, preferred_element_type=f32)

    img = jnp.where(vis_mask, visited_fts, unvisited)
    node_iota = jax.lax.broadcasted_iota(jnp.int32, (M, 1), 0)
    img = jnp.where(node_iota == 0, 0.0, img)

    sid = sid_ref[0]
    s_iota = jax.lax.broadcasted_iota(jnp.int32, (M, MAX_STEPS), 1)
    onehot_step = (sid == s_iota).astype(f32)
    step_emb = jnp.dot(onehot_step, table_ref[...], preferred_element_type=f32)

    h = jnp.dot(pos_ref[0], wpos_ref[...], preferred_element_type=f32) + bpos_ref[...]
    mu = jnp.mean(h, axis=1, keepdims=True)
    var = jnp.mean((h - mu) ** 2, axis=1, keepdims=True)
    ln = (h - mu) / jnp.sqrt(var + EPS) * gam_ref[...] + bet_ref[...]

    out_ref[0] = img + step_emb + ln


@jax.jit
def _encode(split_traj_embeds, split_traj_vp_lens, traj_vpids, traj_cand_vpids,
            gmap_step_ids, gmap_pos_fts, W_pos, b_pos, ln_gamma, ln_beta,
            step_table):
    x = split_traj_embeds.reshape(B, TV, D)
    lens = split_traj_vp_lens.reshape(B, T, 1)
    vpids = traj_vpids.reshape(B, T, 1)
    lens_c = jnp.maximum(split_traj_vp_lens, 1)
    mask_flat = (jnp.arange(V)[None, None, :] < lens_c[:, :, None]).astype(
        jnp.float32).reshape(B, TV, 1)
    cand_flat = traj_cand_vpids.reshape(B, TV, 1)
    sid = gmap_step_ids.reshape(B, M, 1)
    pos = jnp.pad(gmap_pos_fts, ((0, 0), (0, 0), (0, 8 - POSF)))
    wpos = jnp.pad(W_pos, ((0, 8 - POSF), (0, 0)))
    bpos = b_pos.reshape(1, D)
    gam = ln_gamma.reshape(1, D)
    bet = ln_beta.reshape(1, D)

    grid = (B,)
    out = pl.pallas_call(
        _encoder_kernel,
        grid=grid,
        in_specs=[
            pl.BlockSpec((1, TV, D), lambda b: (b, 0, 0)),
            pl.BlockSpec((1, T, 1), lambda b: (b, 0, 0)),
            pl.BlockSpec((1, T, 1), lambda b: (b, 0, 0)),
            pl.BlockSpec((1, TV, 1), lambda b: (b, 0, 0)),
            pl.BlockSpec((1, TV, 1), lambda b: (b, 0, 0)),
            pl.BlockSpec((1, M, 1), lambda b: (b, 0, 0)),
            pl.BlockSpec((1, M, 8), lambda b: (b, 0, 0)),
            pl.BlockSpec((8, D), lambda b: (0, 0)),
            pl.BlockSpec((1, D), lambda b: (0, 0)),
            pl.BlockSpec((1, D), lambda b: (0, 0)),
            pl.BlockSpec((1, D), lambda b: (0, 0)),
            pl.BlockSpec((MAX_STEPS, D), lambda b: (0, 0)),
        ],
        out_specs=pl.BlockSpec((1, M, D), lambda b: (b, 0, 0)),
        out_shape=jax.ShapeDtypeStruct((B, M, D), jnp.float32),
    )(x, lens, vpids, mask_flat, cand_flat, sid, pos, wpos, bpos, gam, bet,
      step_table)
    return out


def kernel(txt_embeds, txt_masks, split_traj_embeds, split_traj_vp_lens,
           traj_vpids, traj_cand_vpids, gmap_vpids, gmap_step_ids,
           gmap_pos_fts, gmap_lens, W_pos, b_pos, ln_gamma, ln_beta,
           step_table):
    return _encode(split_traj_embeds, split_traj_vp_lens, traj_vpids,
                   traj_cand_vpids, gmap_step_ids, gmap_pos_fts, W_pos, b_pos,
                   ln_gamma, ln_beta, step_table)


# speedup vs baseline: 1.3711x; 1.0827x over previous
import functools

import jax
import jax.numpy as jnp
from jax import lax
from jax.experimental import pallas as pl
from jax.experimental.pallas import tpu as pltpu

B, T, V, D = 16, 50, 64, 768
M = 64
POSF = 7
MAX_STEPS = 100
EPS = 1e-12
TV = T * V
ROWS = B * TV

NBUF = 4
CH = 640
TC_ = CH // V
CPB = TV // CH
NCHT = ROWS // CH
TP = 64


def _ring_kernel(x_hbm, mask_hbm, cand_hbm, lens_ref, vpids_ref, sid_ref,
                 pos_ref, wpos_ref, bpos_ref, gam_ref, bet_ref, table_ref,
                 out_ref, b0, b1, b2, b3, mb0, mb1, mb2, mb3,
                 cb0, cb1, cb2, cb3, cand_acc, vis_acc, cnt_acc,
                 s0, s1, s2, s3):
    f32 = jnp.float32
    bufs = (b0, b1, b2, b3)
    mbufs = (mb0, mb1, mb2, mb3)
    cbufs = (cb0, cb1, cb2, cb3)
    sems = (s0, s1, s2, s3)

    def start_all(ch, u):
        pltpu.make_async_copy(x_hbm.at[pl.ds(ch * CH, CH)], bufs[u],
                              sems[u]).start()
        pltpu.make_async_copy(mask_hbm.at[pl.ds(ch * CH, CH)], mbufs[u],
                              sems[u]).start()
        pltpu.make_async_copy(cand_hbm.at[pl.ds(ch * CH, CH)], cbufs[u],
                              sems[u]).start()

    def wait_all(ch, u):
        pltpu.make_async_copy(x_hbm.at[pl.ds(ch * CH, CH)], bufs[u],
                              sems[u]).wait()
        pltpu.make_async_copy(mask_hbm.at[pl.ds(ch * CH, CH)], mbufs[u],
                              sems[u]).wait()
        pltpu.make_async_copy(cand_hbm.at[pl.ds(ch * CH, CH)], cbufs[u],
                              sems[u]).wait()

    for u in range(NBUF):
        start_all(u, u)

    def chunk_body(i, u):
        buf = bufs[u]
        sem = sems[u]
        wait_all(i, u)
        b = i // CPB
        c = i % CPB

        x = buf[...]
        mask_col = mbufs[u][...]
        cand = cbufs[u][...]
        m_iota = jax.lax.broadcasted_iota(jnp.int32, (CH, M), 1)
        onehot = jnp.where(cand == m_iota, mask_col, 0.0)
        part = jax.lax.dot_general(
            onehot, x, (((0,), (0,)), ((), ())), preferred_element_type=f32)
        part_cnt = jnp.sum(onehot, axis=0, keepdims=True)

        masked = x * mask_col
        step_part = jnp.sum(masked.reshape(TC_, V, D), axis=1)

        vp = vpids_ref[pl.ds(b * TP, TP)]
        m_iota_t = jax.lax.broadcasted_iota(jnp.int32, (TP, M), 1)
        t_iota = jax.lax.broadcasted_iota(jnp.int32, (TP, M), 0)
        hit = (vp + 1) == m_iota_t
        tstar = jnp.max(jnp.where(hit, t_iota + 1, 0), axis=0, keepdims=True)
        tstar_col = tstar.reshape(M, 1)
        tloc = tstar_col - 1 - c * TC_
        j_iota = jax.lax.broadcasted_iota(jnp.int32, (M, TC_), 1)
        vis_c = jnp.where((tloc == j_iota) & (tstar_col > 0), 1.0, 0.0)
        vis_part = jnp.dot(vis_c, step_part, preferred_element_type=f32)

        @pl.when(c == 0)
        def _():
            cand_acc[...] = part
            vis_acc[...] = vis_part
            cnt_acc[...] = part_cnt

        @pl.when(c != 0)
        def _():
            cand_acc[...] += part
            vis_acc[...] += vis_part
            cnt_acc[...] += part_cnt

        @pl.when(c == CPB - 1)
        def _():
            lensf = jnp.maximum(lens_ref[pl.ds(b * TP, TP)], 1).astype(f32)
            unvisited = cand_acc[...] / jnp.maximum(
                cnt_acc[...].reshape(M, 1), 1.0)
            t_iota_m = jax.lax.broadcasted_iota(jnp.int32, (M, TP), 1)
            onehot_vis = ((tstar_col - 1) == t_iota_m).astype(f32)
            len_sel = jnp.dot(onehot_vis, lensf, preferred_element_type=f32)
            visited_fts = vis_acc[...] / jnp.maximum(len_sel, 1.0)
            vis_mask = tstar_col > 0
            img = jnp.where(vis_mask, visited_fts, unvisited)
            node_iota = jax.lax.broadcasted_iota(jnp.int32, (M, 1), 0)
            img = jnp.where(node_iota == 0, 0.0, img)

            sid = sid_ref[pl.ds(b * M, M)]
            s_iota = jax.lax.broadcasted_iota(jnp.int32, (M, MAX_STEPS), 1)
            onehot_step = (sid == s_iota).astype(f32)
            step_emb = jnp.dot(onehot_step, table_ref[...],
                               preferred_element_type=f32)

            h = jnp.dot(pos_ref[pl.ds(b * M, M)], wpos_ref[...],
                        preferred_element_type=f32) + bpos_ref[...]
            mu = jnp.mean(h, axis=1, keepdims=True)
            var = jnp.mean((h - mu) ** 2, axis=1, keepdims=True)
            ln = (h - mu) / jnp.sqrt(var + EPS) * gam_ref[...] + bet_ref[...]

            out_ref[pl.ds(b * M, M), :] = img + step_emb + ln

        nxt = i + NBUF

        @pl.when(nxt < NCHT)
        def _():
            start_all(nxt, u)

    def loop_body(it, carry):
        for u in range(NBUF):
            chunk_body(it * NBUF + u, u)
        return carry

    lax.fori_loop(0, NCHT // NBUF, loop_body, 0)


@jax.jit
def _encode(split_traj_embeds, split_traj_vp_lens, traj_vpids, traj_cand_vpids,
            gmap_step_ids, gmap_pos_fts, W_pos, b_pos, ln_gamma, ln_beta,
            step_table):
    x = split_traj_embeds.reshape(ROWS, D)
    lens_c = jnp.maximum(split_traj_vp_lens, 1)
    mask_flat = (jnp.arange(V)[None, None, :] < lens_c[:, :, None]).astype(
        jnp.float32).reshape(ROWS, 1)
    cand_flat = traj_cand_vpids.reshape(ROWS, 1)
    lens_pad = jnp.pad(split_traj_vp_lens, ((0, 0), (0, TP - T)),
                       constant_values=1).reshape(B * TP, 1)
    vpids_pad = jnp.pad(traj_vpids, ((0, 0), (0, TP - T)),
                        constant_values=-2).reshape(B * TP, 1)
    sid = gmap_step_ids.reshape(B * M, 1)
    pos = jnp.pad(gmap_pos_fts, ((0, 0), (0, 0), (0, 8 - POSF))).reshape(
        B * M, 8)
    wpos = jnp.pad(W_pos, ((0, 8 - POSF), (0, 0)))
    bpos = b_pos.reshape(1, D)
    gam = ln_gamma.reshape(1, D)
    bet = ln_beta.reshape(1, D)

    vmem = functools.partial(pl.BlockSpec, memory_space=pltpu.VMEM)
    out = pl.pallas_call(
        _ring_kernel,
        in_specs=[
            pl.BlockSpec(memory_space=pl.ANY),
            pl.BlockSpec(memory_space=pl.ANY),
            pl.BlockSpec(memory_space=pl.ANY),
            vmem(), vmem(), vmem(), vmem(), vmem(), vmem(),
            vmem(), vmem(), vmem(),
        ],
        out_specs=vmem(),
        out_shape=jax.ShapeDtypeStruct((B * M, D), jnp.float32),
        scratch_shapes=[
            pltpu.VMEM((CH, D), jnp.float32),
            pltpu.VMEM((CH, D), jnp.float32),
            pltpu.VMEM((CH, D), jnp.float32),
            pltpu.VMEM((CH, D), jnp.float32),
            pltpu.VMEM((CH, 1), jnp.float32),
            pltpu.VMEM((CH, 1), jnp.float32),
            pltpu.VMEM((CH, 1), jnp.float32),
            pltpu.VMEM((CH, 1), jnp.float32),
            pltpu.VMEM((CH, 1), jnp.int32),
            pltpu.VMEM((CH, 1), jnp.int32),
            pltpu.VMEM((CH, 1), jnp.int32),
            pltpu.VMEM((CH, 1), jnp.int32),
            pltpu.VMEM((M, D), jnp.float32),
            pltpu.VMEM((M, D), jnp.float32),
            pltpu.VMEM((1, M), jnp.float32),
            pltpu.SemaphoreType.DMA,
            pltpu.SemaphoreType.DMA,
            pltpu.SemaphoreType.DMA,
            pltpu.SemaphoreType.DMA,
        ],
    )(x, mask_flat, cand_flat, lens_pad, vpids_pad, sid, pos, wpos, bpos,
      gam, bet, step_table)
    return out.reshape(B, M, D)


def kernel(txt_embeds, txt_masks, split_traj_embeds, split_traj_vp_lens,
           traj_vpids, traj_cand_vpids, gmap_vpids, gmap_step_ids,
           gmap_pos_fts, gmap_lens, W_pos, b_pos, ln_gamma, ln_beta,
           step_table):
    return _encode(split_traj_embeds, split_traj_vp_lens, traj_vpids,
                   traj_cand_vpids, gmap_step_ids, gmap_pos_fts, W_pos, b_pos,
                   ln_gamma, ln_beta, step_table)
